# baseline (speedup 1.0000x reference)
import functools

import jax
import jax.numpy as jnp
from jax import lax
from jax.experimental import pallas as pl
from jax.experimental.pallas import tpu as pltpu
from jax.experimental.pallas import tpu_sc as plsc

_R = 8
_NW = 32
_LANES = 16
_B = 4
_P = 3


def kernel(x, pe_table):
    B, S, D = x.shape
    rows_per_w = S // _NW
    n_chunks = rows_per_w // _R
    n_blocks = (n_chunks - 2) // _P

    pe = pe_table[:S]

    mesh = plsc.VectorSubcoreMesh(core_axis_name="c", subcore_axis_name="s")

    sem = pltpu.SemaphoreType.DMA

    @functools.partial(
        pl.kernel,
        mesh=mesh,
        out_type=jax.ShapeDtypeStruct((B, S, D), jnp.float32),
        scratch_types=(
            [pltpu.VMEM((_R, D), jnp.float32) for _ in range(_P)]
            + [pltpu.VMEM((_B, _R, D), jnp.float32) for _ in range(_P)]
            + [sem] * (3 * _P)
        ),
        compiler_params=pltpu.CompilerParams(use_tc_tiling_on_sc=True),
    )
    def sc_add(x_hbm, pe_hbm, out_hbm, pb0, pb1, pb2, xv0, xv1, xv2,
               ps0, ps1, ps2, gs0, gs1, gs2, ss0, ss1, ss2):
        pebufs = (pb0, pb1, pb2)
        xbufs = (xv0, xv1, xv2)
        psems = (ps0, ps1, ps2)
        gsems = (gs0, gs1, gs2)
        ssems = (ss0, ss1, ss2)

        wid = lax.axis_index("s") * 2 + lax.axis_index("c")
        base = wid * rows_per_w

        def pe_slice(g):
            return pe_hbm.at[pl.ds(base + g * _R, _R)]

        def x_slice(g):
            return x_hbm.at[:, pl.ds(base + g * _R, _R)]

        def out_slice(g):
            return out_hbm.at[:, pl.ds(base + g * _R, _R)]

        def add_chunk(par):
            xbuf = xbufs[par]
            peb = pebufs[par]

            def add_body(i, _):
                r = i >> 4
                cb = (i & 15) * (D // 16)
                for k in range(4):
                    o = cb + k * _LANES
                    vpe = peb[r, pl.ds(o, _LANES)]
                    for b in range(_B):
                        xbuf[b, r, pl.ds(o, _LANES)] = (
                            xbuf[b, r, pl.ds(o, _LANES)] + vpe
                        )
                return 0

            lax.fori_loop(0, _R * D // (_LANES * 4), add_body, 0)

        def chunk_step(g, par, first, pe_pred, gather_pred):
            npar = (par + 2) % _P
            pltpu.make_async_copy(pe_slice(g), pebufs[par], psems[par]).wait()
            pltpu.make_async_copy(x_slice(g), xbufs[par], gsems[par]).wait()
            add_chunk(par)
            pltpu.async_copy(xbufs[par], out_slice(g), ssems[par])
            if not first:
                pltpu.make_async_copy(
                    out_slice(g - 1), xbufs[npar], ssems[npar]).wait()
            if gather_pred is True:
                pltpu.async_copy(x_slice(g + 2), xbufs[npar], gsems[npar])
            elif gather_pred is not None:
                @pl.when(gather_pred)
                def _():
                    pltpu.async_copy(x_slice(g + 2), xbufs[npar], gsems[npar])
            if pe_pred is True:
                pltpu.async_copy(pe_slice(g + _P), pebufs[par], psems[par])
            elif pe_pred is not None:
                @pl.when(pe_pred)
                def _():
                    pltpu.async_copy(pe_slice(g + _P), pebufs[par], psems[par])

        pltpu.async_copy(pe_slice(0), pb0, ps0)
        pltpu.async_copy(pe_slice(1), pb1, ps1)
        pltpu.async_copy(pe_slice(2), pb2, ps2)
        pltpu.async_copy(x_slice(0), xv0, gs0)
        pltpu.async_copy(x_slice(1), xv1, gs1)

        chunk_step(0, 0, first=True, pe_pred=True, gather_pred=True)
        chunk_step(1, 1, first=False, pe_pred=True, gather_pred=True)

        def block_body(gg, _):
            for j in range(_P):
                g = gg * _P + 2 + j
                par = (2 + j) % _P
                more = gg < n_blocks - 1
                chunk_step(g, par, first=False,
                           pe_pred=more, gather_pred=True if j == 0 else more)
            return 0

        lax.fori_loop(0, n_blocks, block_body, 0)

        pltpu.make_async_copy(
            out_slice(n_chunks - 1), xbufs[(n_chunks - 1) % _P],
            ssems[(n_chunks - 1) % _P]).wait()

    return sc_add(x, pe)


# speedup vs baseline: 1.0681x; 1.0681x over previous
import functools

import jax
import jax.numpy as jnp
from jax import lax
from jax.experimental import pallas as pl
from jax.experimental.pallas import tpu as pltpu
from jax.experimental.pallas import tpu_sc as plsc

_R = 8
_NW = 32
_LANES = 16
_B = 4
_P = 3


def kernel(x, pe_table):
    B, S, D = x.shape
    rows_per_w = S // _NW
    n_chunks = rows_per_w // _R
    n_blocks = (n_chunks - 2) // _P

    pe = pe_table[:S]

    mesh = plsc.VectorSubcoreMesh(core_axis_name="c", subcore_axis_name="s")

    vmem = lambda: pltpu.VMEM((_R, D), jnp.float32)
    sem = pltpu.SemaphoreType.DMA

    @functools.partial(
        pl.kernel,
        mesh=mesh,
        out_type=jax.ShapeDtypeStruct((B, S, D), jnp.float32),
        scratch_types=(
            [vmem() for _ in range(_P)]
            + [vmem() for _ in range(_P * _B)]
            + [sem] * (3 * _P)
        ),
        compiler_params=pltpu.CompilerParams(use_tc_tiling_on_sc=True),
    )
    def sc_add(x_hbm, pe_hbm, out_hbm, pb0, pb1, pb2,
               xa0, xa1, xa2, xa3, xb0, xb1, xb2, xb3, xc0, xc1, xc2, xc3,
               ps0, ps1, ps2, gs0, gs1, gs2, ss0, ss1, ss2):
        pebufs = (pb0, pb1, pb2)
        xbufs = ((xa0, xa1, xa2, xa3), (xb0, xb1, xb2, xb3),
                 (xc0, xc1, xc2, xc3))
        psems = (ps0, ps1, ps2)
        gsems = (gs0, gs1, gs2)
        ssems = (ss0, ss1, ss2)

        wid = lax.axis_index("s") * 2 + lax.axis_index("c")
        base = wid * rows_per_w

        def pe_slice(g):
            return pe_hbm.at[pl.ds(base + g * _R, _R)]

        def x_slice(g, b):
            return x_hbm.at[b, pl.ds(base + g * _R, _R)]

        def out_slice(g, b):
            return out_hbm.at[b, pl.ds(base + g * _R, _R)]

        def issue_gathers(g, par):
            for b in range(_B):
                pltpu.async_copy(x_slice(g, b), xbufs[par][b], gsems[par])

        def wait_gathers(g, par):
            for b in range(_B):
                pltpu.make_async_copy(
                    x_slice(g, b), xbufs[par][b], gsems[par]).wait()

        def wait_scatters(g, par):
            for b in range(_B):
                pltpu.make_async_copy(
                    out_slice(g, b), xbufs[par][b], ssems[par]).wait()

        def issue_scatters(g, par):
            for b in range(_B):
                pltpu.async_copy(xbufs[par][b], out_slice(g, b), ssems[par])

        def add_chunk(par):
            bufs = xbufs[par]
            peb = pebufs[par]

            def add_body(i, _):
                r = i >> 4
                cb = (i & 15) * (D // 16)
                for k in range(4):
                    o = cb + k * _LANES
                    vpe = peb[r, pl.ds(o, _LANES)]
                    for b in range(_B):
                        bufs[b][r, pl.ds(o, _LANES)] = (
                            bufs[b][r, pl.ds(o, _LANES)] + vpe
                        )
                return 0

            lax.fori_loop(0, _R * D // (_LANES * 4), add_body, 0)

        def chunk_step(g, par, first, pe_pred, gather_pred):
            npar = (par + 2) % _P
            pltpu.make_async_copy(pe_slice(g), pebufs[par], psems[par]).wait()
            wait_gathers(g, par)
            add_chunk(par)
            issue_scatters(g, par)
            if not first:
                wait_scatters(g - 1, npar)
            if gather_pred is True:
                issue_gathers(g + 2, npar)
            elif gather_pred is not None:
                @pl.when(gather_pred)
                def _():
                    issue_gathers(g + 2, npar)
            if pe_pred is True:
                pltpu.async_copy(pe_slice(g + _P), pebufs[par], psems[par])
            elif pe_pred is not None:
                @pl.when(pe_pred)
                def _():
                    pltpu.async_copy(pe_slice(g + _P), pebufs[par], psems[par])

        pltpu.async_copy(pe_slice(0), pb0, ps0)
        pltpu.async_copy(pe_slice(1), pb1, ps1)
        pltpu.async_copy(pe_slice(2), pb2, ps2)
        issue_gathers(0, 0)
        issue_gathers(1, 1)

        chunk_step(0, 0, first=True, pe_pred=True, gather_pred=True)
        chunk_step(1, 1, first=False, pe_pred=True, gather_pred=True)

        def block_body(gg, _):
            for j in range(_P):
                g = gg * _P + 2 + j
                par = (2 + j) % _P
                more = gg < n_blocks - 1
                chunk_step(g, par, first=False,
                           pe_pred=more, gather_pred=True if j == 0 else more)
            return 0

        lax.fori_loop(0, n_blocks, block_body, 0)

        wait_scatters(n_chunks - 1, (n_chunks - 1) % _P)

    return sc_add(x, pe)
